# baseline (speedup 1.0000x reference)
import functools

import jax
import jax.numpy as jnp
from jax import lax
from jax.experimental import pallas as pl
from jax.experimental.pallas import tpu as pltpu
from jax.experimental.pallas import tpu_sc as plsc

_PAD = 1
_NC, _NS = 2, 16
_NW = _NC * _NS
_D = 64
_BLK = 128
_PACK_BW = 1024


def _pack_table(wT, V):
    def body(wt_ref, out_ref):
        l = lax.broadcasted_iota(jnp.int32, (64, 128), 0)
        c = lax.broadcasted_iota(jnp.int32, (64, 128), 1)
        s_even = (c == 2 * l).astype(jnp.float32)
        s_odd = (c == 2 * l + 1).astype(jnp.float32)
        x = wt_ref[...]
        for i in range(_PACK_BW // 128):
            xs = x[:, i * 128:(i + 1) * 128]
            y1 = lax.dot_general(s_even, xs, (((1,), (1,)), ((), ())),
                                 precision=lax.Precision.HIGHEST)
            y2 = lax.dot_general(s_odd, xs, (((1,), (1,)), ((), ())),
                                 precision=lax.Precision.HIGHEST)
            out_ref[pl.ds(i * 64, 64), :] = jnp.concatenate([y1, y2], axis=1)

    grid = (V + _PACK_BW - 1) // _PACK_BW
    return pl.pallas_call(
        body,
        grid=(grid,),
        in_specs=[pl.BlockSpec((64, _PACK_BW), lambda j: (0, j))],
        out_specs=pl.BlockSpec((_PACK_BW // 2, 128), lambda j: (j, 0)),
        out_shape=jax.ShapeDtypeStruct((V // 2, 128), jnp.float32),
    )(wT)


def _make_gather(T, B, V):
    n_blk = B // _BLK
    assert n_blk == _NW and T % 2 == 0

    mesh = plsc.VectorSubcoreMesh(core_axis_name="c", subcore_axis_name="s")

    @functools.partial(
        pl.kernel,
        out_type=jax.ShapeDtypeStruct((T, 8, n_blk, 1024), jnp.float32),
        mesh=mesh,
        scratch_types=[
            pltpu.VMEM((T, _BLK), jnp.int32),
            pltpu.VMEM((T, _BLK), jnp.int32),
            pltpu.VMEM((2, _BLK, 128), jnp.float32),
            pltpu.VMEM((2, 8, 1024), jnp.float32),
            pltpu.SemaphoreType.DMA((2,)),
            pltpu.SemaphoreType.DMA((2,)),
        ],
        compiler_params=pltpu.CompilerParams(
            use_tc_tiling_on_sc=False, needs_layout_passes=False
        ),
    )
    def gather_kernel(tbl_hbm, idx_hbm, out_hbm, idx_v, line_v, rows_v,
                      blk_v, gsem, osem):
        w = lax.axis_index("s") * _NC + lax.axis_index("c")
        lane = lax.iota(jnp.int32, 16)
        zero16 = jnp.zeros((16,), jnp.float32)

        pltpu.sync_copy(idx_hbm.at[:, pl.ds(w * _BLK, _BLK)], idx_v)

        def lines_body(t, c):
            for g in range(_BLK // 16):
                line_v[t, pl.ds(g * 16, 16)] = (
                    idx_v[t, pl.ds(g * 16, 16)] >> 1
                )
            return c
        lax.fori_loop(0, T, lines_body, 0)

        def fire_gather(k, b):
            pltpu.async_copy(tbl_hbm.at[line_v.at[k]], rows_v.at[b],
                             gsem.at[b])

        def wait_gather(b):
            pltpu.make_async_copy(tbl_hbm.at[pl.ds(0, _BLK)], rows_v.at[b],
                                  gsem.at[b]).wait()

        def fire_out(k, b):
            pltpu.async_copy(blk_v.at[b], out_hbm.at[k].at[:, w], osem.at[b])

        def wait_out(b):
            pltpu.make_async_copy(blk_v.at[b], out_hbm.at[0].at[:, 0],
                                  osem.at[b]).wait()

        def fix_lines(k, b):
            m_any = idx_v[k, pl.ds(0, 16)] == _PAD
            for g in range(1, _BLK // 16):
                m_any = m_any | (idx_v[k, pl.ds(g * 16, 16)] == _PAD)
            n_pad = jnp.sum(m_any.astype(jnp.int32))

            @pl.when(n_pad > 0)
            def _fix():
                for g in range(_BLK // 16):
                    m = idx_v[k, pl.ds(g * 16, 16)] == _PAD
                    b16 = g * 16 + lane

                    def zero_col(cb, cc):
                        plsc.store_scatter(
                            rows_v.at[b],
                            [b16, jnp.full((16,), 0, jnp.int32) + cb],
                            zero16, mask=m)
                        return cc
                    lax.fori_loop(0, 128, zero_col, 0)

        def transpose_item(k, b):
            for g in range(_BLK // 16):
                par = idx_v[k, pl.ds(g * 16, 16)] & 1
                off16 = par * 64
                b16 = g * 16 + lane
                for d in range(_D):
                    v = plsc.load_gather(rows_v.at[b], [b16, off16 + d])
                    blk_v[b, d // 8, pl.ds((d % 8) * 128 + g * 16, 16)] = v

        fire_gather(0, 0)
        fire_gather(1, 1)

        def ring_body(g2, carry):
            for b in range(2):
                k = g2 * 2 + b
                wait_gather(b)
                fix_lines(k, b)

                @pl.when(k >= 2)
                def _drain():
                    wait_out(b)

                transpose_item(k, b)
                fire_out(k, b)

                @pl.when(k + 2 < T)
                def _prefetch():
                    fire_gather(k + 2, b)
            return carry

        lax.fori_loop(0, T // 2, ring_body, 0)
        wait_out(0)
        wait_out(1)

    return gather_kernel


def kernel(indices, weight):
    B, T = indices.shape
    V, D = weight.shape
    assert D == _D
    tbl = _pack_table(weight.T, V)
    idxT = indices.T.astype(jnp.int32)
    out5 = _make_gather(T, B, V)(tbl, idxT)
    out = (
        out5.reshape(T, 8, B // _BLK, 8, 128)
        .transpose(2, 4, 0, 1, 3)
        .reshape(B, T, D)
    )
    return out


# speedup vs baseline: 1.3237x; 1.3237x over previous
import functools

import jax
import jax.numpy as jnp
from jax import lax
from jax.experimental import pallas as pl
from jax.experimental.pallas import tpu as pltpu
from jax.experimental.pallas import tpu_sc as plsc

_PAD = 1
_NC, _NS = 2, 16
_NW = _NC * _NS
_D = 64
_BLK = 128
_PACK_BW = 1024


def _pack_table(wT, V):
    pad_i, pad_c = divmod(_PAD % _PACK_BW, 128)
    pad_blk = _PAD // _PACK_BW

    def body(wt_ref, out_ref):
        j = pl.program_id(0)
        l = lax.broadcasted_iota(jnp.int32, (64, 128), 0)
        c = lax.broadcasted_iota(jnp.int32, (64, 128), 1)
        s_even = (c == 2 * l).astype(jnp.float32)
        s_odd = (c == 2 * l + 1).astype(jnp.float32)
        x = wt_ref[...]
        for i in range(_PACK_BW // 128):
            xs = x[:, i * 128:(i + 1) * 128]
            y1 = lax.dot_general(s_even, xs, (((1,), (1,)), ((), ())),
                                 precision=lax.Precision.HIGHEST)
            y2 = lax.dot_general(s_odd, xs, (((1,), (1,)), ((), ())),
                                 precision=lax.Precision.HIGHEST)
            if i == pad_i:
                l2 = lax.broadcasted_iota(jnp.int32, (64, 64), 0)
                hit = (l2 == pad_c // 2) & (j == pad_blk)
                if _PAD % 2:
                    y2 = jnp.where(hit, 0.0, y2)
                else:
                    y1 = jnp.where(hit, 0.0, y1)
            out_ref[pl.ds(i * 64, 64), :] = jnp.concatenate([y1, y2], axis=1)

    grid = (V + _PACK_BW - 1) // _PACK_BW
    return pl.pallas_call(
        body,
        grid=(grid,),
        in_specs=[pl.BlockSpec((64, _PACK_BW), lambda j: (0, j))],
        out_specs=pl.BlockSpec((_PACK_BW // 2, 128), lambda j: (j, 0)),
        out_shape=jax.ShapeDtypeStruct((V // 2, 128), jnp.float32),
    )(wT)


def _make_gather(T, B, V):
    n_blk = B // _BLK
    assert n_blk == _NW and T % 2 == 0

    mesh = plsc.VectorSubcoreMesh(core_axis_name="c", subcore_axis_name="s")

    @functools.partial(
        pl.kernel,
        out_type=jax.ShapeDtypeStruct((T, 8, n_blk, 1024), jnp.float32),
        mesh=mesh,
        scratch_types=[
            pltpu.VMEM((T, _BLK), jnp.int32),
            pltpu.VMEM((2, _BLK, _D), jnp.float32),
            pltpu.VMEM((2, 8, 1024), jnp.float32),
            pltpu.SemaphoreType.DMA((2,)),
            pltpu.SemaphoreType.DMA((2,)),
        ],
        compiler_params=pltpu.CompilerParams(
            use_tc_tiling_on_sc=False, needs_layout_passes=False
        ),
    )
    def gather_kernel(tbl_hbm, idx_hbm, out_hbm, idx_v, rows_v, blk_v,
                      gsem, osem):
        w = lax.axis_index("s") * _NC + lax.axis_index("c")
        lane = lax.iota(jnp.int32, 16)
        b16s = [g * 16 + lane for g in range(_BLK // 16)]

        pltpu.sync_copy(idx_hbm.at[:, pl.ds(w * _BLK, _BLK)], idx_v)

        def fire_gather(k, b):
            pltpu.async_copy(tbl_hbm.at[idx_v.at[k]], rows_v.at[b],
                             gsem.at[b])

        def wait_gather(b):
            pltpu.make_async_copy(tbl_hbm.at[pl.ds(0, _BLK)], rows_v.at[b],
                                  gsem.at[b]).wait()

        def fire_out(k, b):
            pltpu.async_copy(blk_v.at[b], out_hbm.at[k].at[:, w], osem.at[b])

        def wait_out(b):
            pltpu.make_async_copy(blk_v.at[b], out_hbm.at[0].at[:, 0],
                                  osem.at[b]).wait()

        def transpose_item(k, b):
            for d0 in range(_D):
                d16 = (d0 + lane) & (_D - 1)
                r16 = d16 >> 3
                cb16 = ((d16 & 7) << 7) + lane
                for g in range(_BLK // 16):
                    v = plsc.load_gather(rows_v.at[b], [b16s[g], d16])
                    plsc.store_scatter(blk_v.at[b], [r16, cb16 + g * 16], v)

        fire_gather(0, 0)
        fire_gather(1, 1)

        def ring_body(g2, carry):
            for b in range(2):
                k = g2 * 2 + b
                wait_gather(b)

                @pl.when(k >= 2)
                def _drain():
                    wait_out(b)

                transpose_item(k, b)
                fire_out(k, b)

                @pl.when(k + 2 < T)
                def _prefetch():
                    fire_gather(k + 2, b)
            return carry

        lax.fori_loop(0, T // 2, ring_body, 0)
        wait_out(0)
        wait_out(1)

    return gather_kernel


def kernel(indices, weight):
    B, T = indices.shape
    V, D = weight.shape
    assert D == _D
    tbl = _pack_table(weight.T, V).reshape(V, _D)
    idxT = indices.T.astype(jnp.int32)
    out5 = _make_gather(T, B, V)(tbl, idxT)
    out = (
        out5.reshape(T, 8, B // _BLK, 8, 128)
        .transpose(2, 4, 0, 1, 3)
        .reshape(B, T, D)
    )
    return out


# speedup vs baseline: 1.9576x; 1.4788x over previous
import functools

import jax
import jax.numpy as jnp
from jax import lax
from jax.experimental import pallas as pl
from jax.experimental.pallas import tpu as pltpu
from jax.experimental.pallas import tpu_sc as plsc

_PAD = 1
_NC, _NS = 2, 16
_NW = _NC * _NS
_D = 64
_STREAM = 128
_CHUNK_STREAMS = 2
_CHUNK = _STREAM * _CHUNK_STREAMS
_NBUF = 4
_PF = 2


def _make_gather(B, V):
    per_w = B // _NW
    n_chunks = per_w // _CHUNK
    idx_rows = per_w // _STREAM
    assert per_w % _CHUNK == 0 and n_chunks % _NBUF == 0

    mesh = plsc.VectorSubcoreMesh(core_axis_name="c", subcore_axis_name="s")

    @functools.partial(
        pl.kernel,
        out_type=jax.ShapeDtypeStruct((B, _D), jnp.float32),
        mesh=mesh,
        scratch_types=[
            pltpu.VMEM((idx_rows, _STREAM), jnp.int32),
            pltpu.VMEM((_NBUF, _CHUNK, _D), jnp.float32),
            pltpu.SemaphoreType.DMA((_NBUF,)),
            pltpu.SemaphoreType.DMA((_NBUF,)),
        ],
        compiler_params=pltpu.CompilerParams(
            use_tc_tiling_on_sc=False, needs_layout_passes=False
        ),
    )
    def gather_kernel(weight_hbm, idx_hbm, out_hbm, idx_v, rows_v, gsem, osem):
        wid = lax.axis_index("s") * _NC + lax.axis_index("c")
        lane = lax.iota(jnp.int32, 16)
        zero16 = jnp.zeros((16,), jnp.float32)

        pltpu.sync_copy(idx_hbm.at[pl.ds(wid * idx_rows, idx_rows)], idx_v)

        def fire_gathers(t, b):
            for j in range(_CHUNK_STREAMS):
                pltpu.async_copy(
                    weight_hbm.at[idx_v.at[t * _CHUNK_STREAMS + j]],
                    rows_v.at[b].at[pl.ds(j * _STREAM, _STREAM)],
                    gsem.at[b],
                )

        def wait_gathers(b):
            pltpu.make_async_copy(
                weight_hbm.at[pl.ds(0, _CHUNK)], rows_v.at[b], gsem.at[b]
            ).wait()

        def fire_out(t, b):
            pltpu.async_copy(
                rows_v.at[b],
                out_hbm.at[pl.ds(wid * per_w + t * _CHUNK, _CHUNK)],
                osem.at[b],
            )

        def wait_out(b):
            pltpu.make_async_copy(
                rows_v.at[b], out_hbm.at[pl.ds(0, _CHUNK)], osem.at[b]
            ).wait()

        def fix_padding(t, b):
            row0 = t * _CHUNK_STREAMS
            m_any = idx_v[row0, pl.ds(0, 16)] == _PAD
            for g in range(1, _CHUNK // 16):
                r, o = divmod(g * 16, _STREAM)
                m_any = m_any | (idx_v[row0 + r, pl.ds(o, 16)] == _PAD)
            n_pad = jnp.sum(m_any.astype(jnp.int32))

            @pl.when(n_pad > 0)
            def _fix():
                for r in range(_CHUNK_STREAMS):
                    def fix_group(g, c):
                        idx16 = idx_v[row0 + r, pl.ds(g * 16, 16)]
                        m = idx16 == _PAD
                        @pl.when(jnp.sum(m.astype(jnp.int32)) > 0)
                        def _zero_rows():
                            rows16 = r * _STREAM + g * 16 + lane
                            for col in range(_D):
                                plsc.store_scatter(
                                    rows_v.at[b],
                                    [rows16, jnp.full((16,), col, jnp.int32)],
                                    zero16,
                                    mask=m,
                                )
                        return c
                    lax.fori_loop(0, _STREAM // 16, fix_group, 0)

        for t in range(_PF):
            fire_gathers(t, t % _NBUF)

        def ring_body(g, carry):
            for b in range(_NBUF):
                t = g * _NBUF + b
                wait_gathers(b)
                fix_padding(t, b)
                fire_out(t, b)
                bu = (b + _PF) % _NBUF
                t_pf = t + _PF

                @pl.when(t_pf >= _NBUF)
                def _drain():
                    wait_out(bu)

                @pl.when(t_pf < n_chunks)
                def _prefetch():
                    fire_gathers(t_pf, bu)
            return carry

        lax.fori_loop(0, n_chunks // _NBUF, ring_body, 0)

        for i in range(_PF):
            wait_out((n_chunks - _PF + i) % _NBUF)

    return gather_kernel


def kernel(indices, weight):
    B = indices.size
    V, D = weight.shape
    assert D == _D
    idx2d = indices.reshape(B // _STREAM, _STREAM).astype(jnp.int32)
    out = _make_gather(B, V)(weight, idx2d)
    return out.reshape(indices.shape + (D,))
